# baseline (speedup 1.0000x reference)
import functools

import jax
import jax.numpy as jnp
from jax import lax
from jax.experimental import pallas as pl
from jax.experimental.pallas import tpu as pltpu, tpu_sc as plsc

N = 10000
E = 320000
D1 = 128
D2 = 64
NC = 2
NS = 16
NW = NC * NS
CH = 80
EPW = E // NW
NCHD = EPW // CH
EPT = E // NS
NCHA = EPT // CH
NPAD = 10240
PZ = NPAD // NS
RPT = 624
ZB = 16
RB = 8
G = 7
MAIN = (NCHA // RB) * RB
DSEMS = 5

_MESH = dict(core_axis_name="c", subcore_axis_name="s")


@functools.partial(
    pl.kernel,
    out_type=jax.ShapeDtypeStruct((NC, 2, NPAD), jnp.float32),
    mesh=plsc.VectorSubcoreMesh(**_MESH),
    compiler_params=pltpu.CompilerParams(use_tc_tiling_on_sc=False),
    scratch_types=[
        pltpu.VMEM((NCHD, CH), jnp.int32),
        pltpu.VMEM((NCHD, CH), jnp.int32),
        pltpu.VMEM((CH,), jnp.float32),
        pltpu.VMEM((PZ,), jnp.float32),
        pltpu.VMEM_SHARED((NPAD,), jnp.float32),
        pltpu.VMEM_SHARED((NPAD,), jnp.float32),
        [pltpu.SemaphoreType.DMA for _ in range(DSEMS)],
        [pltpu.SemaphoreType.DMA for _ in range(DSEMS)],
    ],
)
def _deg(src_hbm, dst_hbm, out_hbm, sidx_v, didx_v, ones_v, zero_v,
         dego_sh, degi_sh, osem, isem):
    c = lax.axis_index("c")
    s = lax.axis_index("s")
    wid = s * NC + c

    for i in range(CH // 16):
        ones_v[pl.ds(i * 16, 16)] = jnp.ones((16,), jnp.float32)

    def _zb(i, _):
        zero_v[pl.ds(i * 16, 16)] = jnp.zeros((16,), jnp.float32)
        return ()
    lax.fori_loop(0, PZ // 16, _zb, ())

    base = s * PZ
    pltpu.sync_copy(zero_v, dego_sh.at[pl.ds(base, PZ)])
    pltpu.sync_copy(zero_v, degi_sh.at[pl.ds(base, PZ)])

    pltpu.sync_copy(src_hbm.at[wid], sidx_v)
    pltpu.sync_copy(dst_hbm.at[wid], didx_v)
    plsc.subcore_barrier()

    def _body(i, _):
        for j in range(DSEMS):
            k = i * DSEMS + j
            @pl.when(k >= DSEMS)
            def _():
                pltpu.make_async_copy(ones_v, dego_sh.at[sidx_v.at[k]],
                                      osem[j]).wait()
                pltpu.make_async_copy(ones_v, degi_sh.at[didx_v.at[k]],
                                      isem[j]).wait()
            pltpu.async_copy(ones_v, dego_sh.at[sidx_v.at[k]], osem[j],
                             add=True)
            pltpu.async_copy(ones_v, degi_sh.at[didx_v.at[k]], isem[j],
                             add=True)
        return ()
    lax.fori_loop(0, NCHD // DSEMS, _body, ())

    for t in range(NCHD - DSEMS, NCHD):
        b = t % DSEMS
        pltpu.make_async_copy(ones_v, dego_sh.at[sidx_v.at[t]], osem[b]).wait()
        pltpu.make_async_copy(ones_v, degi_sh.at[didx_v.at[t]], isem[b]).wait()

    plsc.subcore_barrier()
    pltpu.sync_copy(dego_sh.at[pl.ds(base, PZ)], out_hbm.at[c, 0, pl.ds(base, PZ)])
    pltpu.sync_copy(degi_sh.at[pl.ds(base, PZ)], out_hbm.at[c, 1, pl.ds(base, PZ)])


def _make_agg(HALF):
    @functools.partial(
        pl.kernel,
        out_type=jax.ShapeDtypeStruct((NC, N, HALF), jnp.float32),
        mesh=plsc.VectorSubcoreMesh(**_MESH),
        compiler_params=pltpu.CompilerParams(use_tc_tiling_on_sc=False),
        scratch_types=[
            pltpu.VMEM((NCHA, CH), jnp.int32),
            pltpu.VMEM((NCHA, CH), jnp.int32),
            [pltpu.VMEM((CH, HALF), jnp.float32) for _ in range(RB)],
            pltpu.VMEM((ZB, HALF), jnp.float32),
            pltpu.VMEM_SHARED((N, HALF), jnp.float32),
            [pltpu.SemaphoreType.DMA for _ in range(RB)],
            [pltpu.SemaphoreType.DMA for _ in range(RB)],
        ],
    )
    def _agg(h_hbm, src_hbm, dst_hbm, out_hbm, sidx_v, didx_v, rows,
             zero_v, agg_sh, gsem, ssem):
        c = lax.axis_index("c")
        s = lax.axis_index("s")

        def _zb(i, _):
            for j in range(HALF // 16):
                zero_v[i, pl.ds(j * 16, 16)] = jnp.zeros((16,), jnp.float32)
            return ()
        lax.fori_loop(0, ZB, _zb, ())

        def _zs(j, _):
            pltpu.sync_copy(zero_v, agg_sh.at[pl.ds(s * RPT + j * ZB, ZB)])
            return ()
        lax.fori_loop(0, RPT // ZB, _zs, ())
        @pl.when(s == NS - 1)
        def _():
            pltpu.sync_copy(zero_v, agg_sh.at[pl.ds(NS * RPT, ZB)])

        pltpu.sync_copy(src_hbm.at[s], sidx_v)
        pltpu.sync_copy(dst_hbm.at[s], didx_v)
        plsc.subcore_barrier()

        hc = h_hbm.at[c]

        def _gissue(k, b):
            pltpu.async_copy(hc.at[sidx_v.at[k]], rows[b], gsem[b])

        def _gwait(k, b):
            pltpu.make_async_copy(hc.at[sidx_v.at[k]], rows[b], gsem[b]).wait()

        def _sissue(k, b):
            pltpu.async_copy(rows[b], agg_sh.at[didx_v.at[k]], ssem[b],
                             add=True)

        def _swait(k, b):
            pltpu.make_async_copy(rows[b], agg_sh.at[didx_v.at[k]],
                                  ssem[b]).wait()

        for kk in range(G):
            _gissue(kk, kk)

        def _body(i, _):
            for j in range(RB):
                k = i * RB + j
                bg = (j + G) % RB
                @pl.when(k + G < NCHA)
                def _():
                    @pl.when(k + G >= RB)
                    def _():
                        _swait(k + G - RB, bg)
                    _gissue(k + G, bg)
                _gwait(k, j)
                _sissue(k, j)
            return ()
        lax.fori_loop(0, NCHA // RB, _body, ())

        for t in range(MAIN, NCHA):
            _gwait(t, t % RB)
            _sissue(t, t % RB)

        for t in range(NCHA - RB, NCHA):
            _swait(t, t % RB)

        plsc.subcore_barrier()
        pltpu.sync_copy(agg_sh.at[pl.ds(s * RPT, RPT)],
                        out_hbm.at[c].at[pl.ds(s * RPT, RPT)])
        @pl.when(s == NS - 1)
        def _():
            pltpu.sync_copy(agg_sh.at[pl.ds(NS * RPT, ZB)],
                            out_hbm.at[c].at[pl.ds(NS * RPT, ZB)])
    return _agg


_agg1 = _make_agg(D1 // NC)

_H2 = D2 // NC


@functools.partial(
    pl.kernel,
    out_type=jax.ShapeDtypeStruct((N, D2), jnp.float32),
    mesh=plsc.VectorSubcoreMesh(**_MESH),
    compiler_params=pltpu.CompilerParams(use_tc_tiling_on_sc=False),
    scratch_types=[
        pltpu.VMEM((NCHA, CH), jnp.int32),
        pltpu.VMEM((NCHA, CH), jnp.int32),
        [pltpu.VMEM((CH, _H2), jnp.float32) for _ in range(RB)],
        pltpu.VMEM((ZB, _H2), jnp.float32),
        pltpu.VMEM((PZ, _H2), jnp.float32),
        pltpu.VMEM((PZ + 16,), jnp.float32),
        pltpu.VMEM((_H2,), jnp.float32),
        pltpu.VMEM_SHARED((N, _H2), jnp.float32),
        [pltpu.SemaphoreType.DMA for _ in range(RB)],
        [pltpu.SemaphoreType.DMA for _ in range(RB)],
    ],
)
def _agg_final(h_hbm, src_hbm, dst_hbm, ni_hbm, b2_hbm, out_hbm,
               sidx_v, didx_v, rows, zero_v, abuf, nbuf, bbuf,
               agg_sh, gsem, ssem):
    c = lax.axis_index("c")
    s = lax.axis_index("s")

    def _zb(i, _):
        for j in range(_H2 // 16):
            zero_v[i, pl.ds(j * 16, 16)] = jnp.zeros((16,), jnp.float32)
        return ()
    lax.fori_loop(0, ZB, _zb, ())

    def _zs(j, _):
        pltpu.sync_copy(zero_v, agg_sh.at[pl.ds(s * RPT + j * ZB, ZB)])
        return ()
    lax.fori_loop(0, RPT // ZB, _zs, ())
    @pl.when(s == NS - 1)
    def _():
        pltpu.sync_copy(zero_v, agg_sh.at[pl.ds(NS * RPT, ZB)])

    pltpu.sync_copy(src_hbm.at[s], sidx_v)
    pltpu.sync_copy(dst_hbm.at[s], didx_v)
    pltpu.sync_copy(b2_hbm.at[pl.ds(c * _H2, _H2)], bbuf)
    plsc.subcore_barrier()

    hc = h_hbm.at[c]

    def _gissue(k, b):
        pltpu.async_copy(hc.at[sidx_v.at[k]], rows[b], gsem[b])

    def _gwait(k, b):
        pltpu.make_async_copy(hc.at[sidx_v.at[k]], rows[b], gsem[b]).wait()

    def _sissue(k, b):
        pltpu.async_copy(rows[b], agg_sh.at[didx_v.at[k]], ssem[b], add=True)

    def _swait(k, b):
        pltpu.make_async_copy(rows[b], agg_sh.at[didx_v.at[k]],
                              ssem[b]).wait()

    for kk in range(G):
        _gissue(kk, kk)

    def _body(i, _):
        for j in range(RB):
            k = i * RB + j
            bg = (j + G) % RB
            @pl.when(k + G < NCHA)
            def _():
                @pl.when(k + G >= RB)
                def _():
                    _swait(k + G - RB, bg)
                _gissue(k + G, bg)
            _gwait(k, j)
            _sissue(k, j)
        return ()
    lax.fori_loop(0, NCHA // RB, _body, ())

    for t in range(MAIN, NCHA):
        _gwait(t, t % RB)
        _sissue(t, t % RB)

    for t in range(NCHA - RB, NCHA):
        _swait(t, t % RB)

    plsc.subcore_barrier()

    last = s == NS - 1

    @pl.when(jnp.logical_not(last))
    def _():
        pltpu.sync_copy(agg_sh.at[pl.ds(s * RPT, RPT)],
                        abuf.at[pl.ds(0, RPT)])
        pltpu.sync_copy(ni_hbm.at[pl.ds(s * RPT, RPT)],
                        nbuf.at[pl.ds(0, RPT)])

    @pl.when(last)
    def _():
        pltpu.sync_copy(agg_sh.at[pl.ds(NS * RPT - RPT, PZ)], abuf)
        pltpu.sync_copy(ni_hbm.at[pl.ds(NS * RPT - RPT, PZ)],
                        nbuf.at[pl.ds(0, PZ)])

    def _post(r, _):
        ni = nbuf[pl.ds(r, 16)][0]
        for q in range(_H2 // 16):
            sl = pl.ds(q * 16, 16)
            abuf[r, sl] = abuf[r, sl] * ni + bbuf[sl]
        return ()
    lax.fori_loop(0, RPT, _post, ())
    @pl.when(last)
    def _():
        lax.fori_loop(RPT, PZ, _post, ())

    @pl.when(jnp.logical_not(last))
    def _():
        pltpu.sync_copy(abuf.at[pl.ds(0, RPT)],
                        out_hbm.at[pl.ds(s * RPT, RPT), pl.ds(c * _H2, _H2)])

    @pl.when(last)
    def _():
        pltpu.sync_copy(abuf,
                        out_hbm.at[pl.ds(NS * RPT - RPT, PZ),
                                   pl.ds(c * _H2, _H2)])


BM = 2000
H1 = D1 // NC
H2 = D2 // NC


def _layer1_tc(x, W1, deg):
    def body(deg_ref, x_ref, w_ref, h_ref, nrm_ref, ni_ref):
        d = deg_ref[0] + deg_ref[1]
        n = lax.rsqrt(jnp.maximum(d[:, :N], 1.0))
        nrm = n.T
        nrm_ref[...] = nrm
        ni_ref[...] = n[1]
        h = jnp.dot(x_ref[...] * nrm[:, 0:1], w_ref[...],
                    preferred_element_type=jnp.float32)
        h_ref[0] = h[:, :H1]
        h_ref[1] = h[:, H1:]
    return pl.pallas_call(
        body,
        out_shape=(jax.ShapeDtypeStruct((NC, N, H1), jnp.float32),
                   jax.ShapeDtypeStruct((N, 2), jnp.float32),
                   jax.ShapeDtypeStruct((N,), jnp.float32)),
    )(deg, x, W1)


def _layer2_tc(agg, b1, W2, norms):
    def body(agg_ref, b_ref, w_ref, nrm_ref, h_ref):
        no = nrm_ref[:, 0:1]
        ni = nrm_ref[:, 1:2]
        t0 = jnp.maximum(agg_ref[0] * ni + b_ref[:, :H1], 0.0) * no
        t1 = jnp.maximum(agg_ref[1] * ni + b_ref[:, H1:], 0.0) * no
        h = (jnp.dot(t0, w_ref[:H1], preferred_element_type=jnp.float32)
             + jnp.dot(t1, w_ref[H1:], preferred_element_type=jnp.float32))
        h_ref[0] = h[:, :H2]
        h_ref[1] = h[:, H2:]
    return pl.pallas_call(
        body,
        grid=(N // BM,),
        in_specs=[
            pl.BlockSpec((NC, BM, H1), lambda i: (0, i, 0)),
            pl.BlockSpec((1, D1), lambda i: (0, 0)),
            pl.BlockSpec((D1, D2), lambda i: (0, 0)),
            pl.BlockSpec((BM, 2), lambda i: (i, 0)),
        ],
        out_specs=pl.BlockSpec((NC, BM, H2), lambda i: (0, i, 0)),
        out_shape=jax.ShapeDtypeStruct((NC, N, H2), jnp.float32),
    )(agg, b1.reshape(1, D1), W2, norms)


def kernel(x, edge_index, W1, b1, W2, b2):
    srcD = edge_index[0].reshape(NW, NCHD, CH)
    dstD = edge_index[1].reshape(NW, NCHD, CH)
    srcA = edge_index[0].reshape(NS, NCHA, CH)
    dstA = edge_index[1].reshape(NS, NCHA, CH)

    deg = _deg(srcD, dstD)
    h1, norms, ni = _layer1_tc(x, W1, deg)
    agg1 = _agg1(h1, srcA, dstA)
    h2 = _layer2_tc(agg1, b1, W2, norms)
    return _agg_final(h2, srcA, dstA, ni, b2)


# speedup vs baseline: 1.0068x; 1.0068x over previous
import functools

import jax
import jax.numpy as jnp
from jax import lax
from jax.experimental import pallas as pl
from jax.experimental.pallas import tpu as pltpu, tpu_sc as plsc

N = 10000
E = 320000
D1 = 128
D2 = 64
NC = 2
NS = 16
NW = NC * NS
CH = 80
EPW = E // NW
NCHD = EPW // CH
EPT = E // NS
NCHA = EPT // CH
NPAD = 10240
PZ = NPAD // NS
RPT = 624
ZB = 16
RB = 8
G = 6
MAIN = (NCHA // RB) * RB
DSEMS = 5

_MESH = dict(core_axis_name="c", subcore_axis_name="s")


@functools.partial(
    pl.kernel,
    out_type=jax.ShapeDtypeStruct((NC, 2, NPAD), jnp.float32),
    mesh=plsc.VectorSubcoreMesh(**_MESH),
    compiler_params=pltpu.CompilerParams(use_tc_tiling_on_sc=False),
    scratch_types=[
        pltpu.VMEM((NCHD, CH), jnp.int32),
        pltpu.VMEM((NCHD, CH), jnp.int32),
        pltpu.VMEM((CH,), jnp.float32),
        pltpu.VMEM((PZ,), jnp.float32),
        pltpu.VMEM_SHARED((NPAD,), jnp.float32),
        pltpu.VMEM_SHARED((NPAD,), jnp.float32),
        [pltpu.SemaphoreType.DMA for _ in range(DSEMS)],
        [pltpu.SemaphoreType.DMA for _ in range(DSEMS)],
    ],
)
def _deg(src_hbm, dst_hbm, out_hbm, sidx_v, didx_v, ones_v, zero_v,
         dego_sh, degi_sh, osem, isem):
    c = lax.axis_index("c")
    s = lax.axis_index("s")
    wid = s * NC + c

    for i in range(CH // 16):
        ones_v[pl.ds(i * 16, 16)] = jnp.ones((16,), jnp.float32)

    def _zb(i, _):
        zero_v[pl.ds(i * 16, 16)] = jnp.zeros((16,), jnp.float32)
        return ()
    lax.fori_loop(0, PZ // 16, _zb, ())

    base = s * PZ
    pltpu.sync_copy(zero_v, dego_sh.at[pl.ds(base, PZ)])
    pltpu.sync_copy(zero_v, degi_sh.at[pl.ds(base, PZ)])

    pltpu.sync_copy(src_hbm.at[wid], sidx_v)
    pltpu.sync_copy(dst_hbm.at[wid], didx_v)
    plsc.subcore_barrier()

    def _body(i, _):
        for j in range(DSEMS):
            k = i * DSEMS + j
            @pl.when(k >= DSEMS)
            def _():
                pltpu.make_async_copy(ones_v, dego_sh.at[sidx_v.at[k]],
                                      osem[j]).wait()
                pltpu.make_async_copy(ones_v, degi_sh.at[didx_v.at[k]],
                                      isem[j]).wait()
            pltpu.async_copy(ones_v, dego_sh.at[sidx_v.at[k]], osem[j],
                             add=True)
            pltpu.async_copy(ones_v, degi_sh.at[didx_v.at[k]], isem[j],
                             add=True)
        return ()
    lax.fori_loop(0, NCHD // DSEMS, _body, ())

    for t in range(NCHD - DSEMS, NCHD):
        b = t % DSEMS
        pltpu.make_async_copy(ones_v, dego_sh.at[sidx_v.at[t]], osem[b]).wait()
        pltpu.make_async_copy(ones_v, degi_sh.at[didx_v.at[t]], isem[b]).wait()

    plsc.subcore_barrier()
    pltpu.sync_copy(dego_sh.at[pl.ds(base, PZ)], out_hbm.at[c, 0, pl.ds(base, PZ)])
    pltpu.sync_copy(degi_sh.at[pl.ds(base, PZ)], out_hbm.at[c, 1, pl.ds(base, PZ)])


def _make_agg(HALF):
    @functools.partial(
        pl.kernel,
        out_type=jax.ShapeDtypeStruct((NC, N, HALF), jnp.float32),
        mesh=plsc.VectorSubcoreMesh(**_MESH),
        compiler_params=pltpu.CompilerParams(use_tc_tiling_on_sc=False),
        scratch_types=[
            pltpu.VMEM((NCHA, CH), jnp.int32),
            pltpu.VMEM((NCHA, CH), jnp.int32),
            [pltpu.VMEM((CH, HALF), jnp.float32) for _ in range(RB)],
            pltpu.VMEM((ZB, HALF), jnp.float32),
            pltpu.VMEM_SHARED((N, HALF), jnp.float32),
            [pltpu.SemaphoreType.DMA for _ in range(RB)],
            [pltpu.SemaphoreType.DMA for _ in range(RB)],
        ],
    )
    def _agg(h_hbm, src_hbm, dst_hbm, out_hbm, sidx_v, didx_v, rows,
             zero_v, agg_sh, gsem, ssem):
        c = lax.axis_index("c")
        s = lax.axis_index("s")

        def _zb(i, _):
            for j in range(HALF // 16):
                zero_v[i, pl.ds(j * 16, 16)] = jnp.zeros((16,), jnp.float32)
            return ()
        lax.fori_loop(0, ZB, _zb, ())

        def _zs(j, _):
            pltpu.sync_copy(zero_v, agg_sh.at[pl.ds(s * RPT + j * ZB, ZB)])
            return ()
        lax.fori_loop(0, RPT // ZB, _zs, ())
        @pl.when(s == NS - 1)
        def _():
            pltpu.sync_copy(zero_v, agg_sh.at[pl.ds(NS * RPT, ZB)])

        pltpu.sync_copy(src_hbm.at[s], sidx_v)
        pltpu.sync_copy(dst_hbm.at[s], didx_v)
        plsc.subcore_barrier()

        hc = h_hbm.at[c]

        def _gissue(k, b):
            pltpu.async_copy(hc.at[sidx_v.at[k]], rows[b], gsem[b])

        def _gwait(k, b):
            pltpu.make_async_copy(hc.at[sidx_v.at[k]], rows[b], gsem[b]).wait()

        def _sissue(k, b):
            pltpu.async_copy(rows[b], agg_sh.at[didx_v.at[k]], ssem[b],
                             add=True)

        def _swait(k, b):
            pltpu.make_async_copy(rows[b], agg_sh.at[didx_v.at[k]],
                                  ssem[b]).wait()

        for kk in range(G):
            _gissue(kk, kk)

        def _body(i, _):
            for j in range(RB):
                k = i * RB + j
                bg = (j + G) % RB
                @pl.when(k + G < NCHA)
                def _():
                    @pl.when(k + G >= RB)
                    def _():
                        _swait(k + G - RB, bg)
                    _gissue(k + G, bg)
                _gwait(k, j)
                _sissue(k, j)
            return ()
        lax.fori_loop(0, NCHA // RB, _body, ())

        for t in range(MAIN, NCHA):
            _gwait(t, t % RB)
            _sissue(t, t % RB)

        for t in range(NCHA - RB, NCHA):
            _swait(t, t % RB)

        plsc.subcore_barrier()
        pltpu.sync_copy(agg_sh.at[pl.ds(s * RPT, RPT)],
                        out_hbm.at[c].at[pl.ds(s * RPT, RPT)])
        @pl.when(s == NS - 1)
        def _():
            pltpu.sync_copy(agg_sh.at[pl.ds(NS * RPT, ZB)],
                            out_hbm.at[c].at[pl.ds(NS * RPT, ZB)])
    return _agg


_agg1 = _make_agg(D1 // NC)

_H2 = D2 // NC


@functools.partial(
    pl.kernel,
    out_type=jax.ShapeDtypeStruct((N, D2), jnp.float32),
    mesh=plsc.VectorSubcoreMesh(**_MESH),
    compiler_params=pltpu.CompilerParams(use_tc_tiling_on_sc=False),
    scratch_types=[
        pltpu.VMEM((NCHA, CH), jnp.int32),
        pltpu.VMEM((NCHA, CH), jnp.int32),
        [pltpu.VMEM((CH, _H2), jnp.float32) for _ in range(RB)],
        pltpu.VMEM((ZB, _H2), jnp.float32),
        pltpu.VMEM((PZ, _H2), jnp.float32),
        pltpu.VMEM((PZ + 16,), jnp.float32),
        pltpu.VMEM((_H2,), jnp.float32),
        pltpu.VMEM_SHARED((N, _H2), jnp.float32),
        [pltpu.SemaphoreType.DMA for _ in range(RB)],
        [pltpu.SemaphoreType.DMA for _ in range(RB)],
    ],
)
def _agg_final(h_hbm, src_hbm, dst_hbm, ni_hbm, b2_hbm, out_hbm,
               sidx_v, didx_v, rows, zero_v, abuf, nbuf, bbuf,
               agg_sh, gsem, ssem):
    c = lax.axis_index("c")
    s = lax.axis_index("s")

    def _zb(i, _):
        for j in range(_H2 // 16):
            zero_v[i, pl.ds(j * 16, 16)] = jnp.zeros((16,), jnp.float32)
        return ()
    lax.fori_loop(0, ZB, _zb, ())

    def _zs(j, _):
        pltpu.sync_copy(zero_v, agg_sh.at[pl.ds(s * RPT + j * ZB, ZB)])
        return ()
    lax.fori_loop(0, RPT // ZB, _zs, ())
    @pl.when(s == NS - 1)
    def _():
        pltpu.sync_copy(zero_v, agg_sh.at[pl.ds(NS * RPT, ZB)])

    pltpu.sync_copy(src_hbm.at[s], sidx_v)
    pltpu.sync_copy(dst_hbm.at[s], didx_v)
    pltpu.sync_copy(b2_hbm.at[pl.ds(c * _H2, _H2)], bbuf)
    plsc.subcore_barrier()

    hc = h_hbm.at[c]

    def _gissue(k, b):
        pltpu.async_copy(hc.at[sidx_v.at[k]], rows[b], gsem[b])

    def _gwait(k, b):
        pltpu.make_async_copy(hc.at[sidx_v.at[k]], rows[b], gsem[b]).wait()

    def _sissue(k, b):
        pltpu.async_copy(rows[b], agg_sh.at[didx_v.at[k]], ssem[b], add=True)

    def _swait(k, b):
        pltpu.make_async_copy(rows[b], agg_sh.at[didx_v.at[k]],
                              ssem[b]).wait()

    for kk in range(G):
        _gissue(kk, kk)

    def _body(i, _):
        for j in range(RB):
            k = i * RB + j
            bg = (j + G) % RB
            @pl.when(k + G < NCHA)
            def _():
                @pl.when(k + G >= RB)
                def _():
                    _swait(k + G - RB, bg)
                _gissue(k + G, bg)
            _gwait(k, j)
            _sissue(k, j)
        return ()
    lax.fori_loop(0, NCHA // RB, _body, ())

    for t in range(MAIN, NCHA):
        _gwait(t, t % RB)
        _sissue(t, t % RB)

    for t in range(NCHA - RB, NCHA):
        _swait(t, t % RB)

    plsc.subcore_barrier()

    last = s == NS - 1

    @pl.when(jnp.logical_not(last))
    def _():
        pltpu.sync_copy(agg_sh.at[pl.ds(s * RPT, RPT)],
                        abuf.at[pl.ds(0, RPT)])
        pltpu.sync_copy(ni_hbm.at[pl.ds(s * RPT, RPT)],
                        nbuf.at[pl.ds(0, RPT)])

    @pl.when(last)
    def _():
        pltpu.sync_copy(agg_sh.at[pl.ds(NS * RPT - RPT, PZ)], abuf)
        pltpu.sync_copy(ni_hbm.at[pl.ds(NS * RPT - RPT, PZ)],
                        nbuf.at[pl.ds(0, PZ)])

    def _post(r, _):
        ni = nbuf[pl.ds(r, 16)][0]
        for q in range(_H2 // 16):
            sl = pl.ds(q * 16, 16)
            abuf[r, sl] = abuf[r, sl] * ni + bbuf[sl]
        return ()
    lax.fori_loop(0, RPT, _post, ())
    @pl.when(last)
    def _():
        lax.fori_loop(RPT, PZ, _post, ())

    @pl.when(jnp.logical_not(last))
    def _():
        pltpu.sync_copy(abuf.at[pl.ds(0, RPT)],
                        out_hbm.at[pl.ds(s * RPT, RPT), pl.ds(c * _H2, _H2)])

    @pl.when(last)
    def _():
        pltpu.sync_copy(abuf,
                        out_hbm.at[pl.ds(NS * RPT - RPT, PZ),
                                   pl.ds(c * _H2, _H2)])


BM = 2000
H1 = D1 // NC
H2 = D2 // NC


def _layer1_tc(x, W1, deg):
    def body(deg_ref, x_ref, w_ref, h_ref, nrm_ref, ni_ref):
        d = deg_ref[0] + deg_ref[1]
        n = lax.rsqrt(jnp.maximum(d[:, :N], 1.0))
        nrm = n.T
        nrm_ref[...] = nrm
        ni_ref[...] = n[1]
        h = jnp.dot(x_ref[...] * nrm[:, 0:1], w_ref[...],
                    preferred_element_type=jnp.float32)
        h_ref[0] = h[:, :H1]
        h_ref[1] = h[:, H1:]
    return pl.pallas_call(
        body,
        out_shape=(jax.ShapeDtypeStruct((NC, N, H1), jnp.float32),
                   jax.ShapeDtypeStruct((N, 2), jnp.float32),
                   jax.ShapeDtypeStruct((N,), jnp.float32)),
    )(deg, x, W1)


def _layer2_tc(agg, b1, W2, norms):
    def body(agg_ref, b_ref, w_ref, nrm_ref, h_ref):
        no = nrm_ref[:, 0:1]
        ni = nrm_ref[:, 1:2]
        t0 = jnp.maximum(agg_ref[0] * ni + b_ref[:, :H1], 0.0) * no
        t1 = jnp.maximum(agg_ref[1] * ni + b_ref[:, H1:], 0.0) * no
        h = (jnp.dot(t0, w_ref[:H1], preferred_element_type=jnp.float32)
             + jnp.dot(t1, w_ref[H1:], preferred_element_type=jnp.float32))
        h_ref[0] = h[:, :H2]
        h_ref[1] = h[:, H2:]
    return pl.pallas_call(
        body,
        grid=(N // BM,),
        in_specs=[
            pl.BlockSpec((NC, BM, H1), lambda i: (0, i, 0)),
            pl.BlockSpec((1, D1), lambda i: (0, 0)),
            pl.BlockSpec((D1, D2), lambda i: (0, 0)),
            pl.BlockSpec((BM, 2), lambda i: (i, 0)),
        ],
        out_specs=pl.BlockSpec((NC, BM, H2), lambda i: (0, i, 0)),
        out_shape=jax.ShapeDtypeStruct((NC, N, H2), jnp.float32),
    )(agg, b1.reshape(1, D1), W2, norms)


def kernel(x, edge_index, W1, b1, W2, b2):
    srcD = edge_index[0].reshape(NW, NCHD, CH)
    dstD = edge_index[1].reshape(NW, NCHD, CH)
    srcA = edge_index[0].reshape(NS, NCHA, CH)
    dstA = edge_index[1].reshape(NS, NCHA, CH)

    deg = _deg(srcD, dstD)
    h1, norms, ni = _layer1_tc(x, W1, deg)
    agg1 = _agg1(h1, srcA, dstA)
    h2 = _layer2_tc(agg1, b1, W2, norms)
    return _agg_final(h2, srcA, dstA, ni, b2)
